# baseline (speedup 1.0000x reference)
import functools

import jax
import jax.numpy as jnp
from jax import lax
from jax.experimental import pallas as pl
from jax.experimental.pallas import tpu as pltpu
from jax.experimental.pallas import tpu_sc as plsc

VOCAB = 24
LANES = 16
N_CORES = 2
N_SUBCORES = 16
N_WORKERS = N_CORES * N_SUBCORES
BGROUP = 8
KSTRIDE = 32
LANE_STRIDE = VOCAB * KSTRIDE + 1
REP_WORDS = LANE_STRIDE * LANES
SC_BATCH = 32
TC_SBLK = 2048


def _make_sc_kernel(sc_batch: int, seq: int):
  n_sblocks = N_WORKERS // (sc_batch // BGROUP)
  sblk = seq // n_sblocks
  mesh = plsc.VectorSubcoreMesh(core_axis_name="c", subcore_axis_name="s")

  scratch = (
      [pltpu.VMEM((VOCAB, sblk), jnp.float32)] * 2
      + [
          pltpu.VMEM((BGROUP, sblk), jnp.int32),
          pltpu.VMEM((VOCAB * VOCAB,), jnp.float32),
          pltpu.VMEM((REP_WORDS,), jnp.float32),
          pltpu.VMEM((LANES,), jnp.float32),
      ]
      + [pltpu.SemaphoreType.DMA] * 3
  )

  @functools.partial(
      pl.kernel,
      out_type=jax.ShapeDtypeStruct((N_WORKERS * LANES,), jnp.float32),
      mesh=mesh,
      scratch_types=scratch,
      compiler_params=pltpu.CompilerParams(
          use_tc_tiling_on_sc=True, needs_layout_passes=False),
  )
  def blosum_sc(yt_hbm, yp_hbm, b_hbm, out_hbm,
                pred0, pred1, idx_v, b_v, brep_v, acc_v,
                sem0, sem1, sem_i):
    pred_b = (pred0, pred1)
    sems = (sem0, sem1)
    cid = lax.axis_index("c")
    sid = lax.axis_index("s")
    wid = sid * N_CORES + cid
    bg = wid // n_sblocks
    s0 = (wid % n_sblocks) * sblk

    d_idx = pltpu.async_copy(
        yt_hbm.at[pl.ds(bg * BGROUP, BGROUP), pl.ds(s0, sblk)], idx_v, sem_i)

    def start(r, buf):
      return pltpu.async_copy(
          yp_hbm.at[bg * BGROUP + r].at[:, pl.ds(s0, sblk)],
          pred_b[buf], sems[buf])

    pend = [start(0, 0), start(1, 1)]

    pltpu.sync_copy(b_hbm, b_v)
    for k in range(VOCAB):
      lo = b_v[pl.ds(k * VOCAB, LANES)]
      hi = b_v[pl.ds(k * VOCAB + VOCAB - LANES, LANES)]
      for l in range(LANES):
        off = l * LANE_STRIDE + k * KSTRIDE
        brep_v[pl.ds(off, LANES)] = lo
        brep_v[pl.ds(off + VOCAB - LANES, LANES)] = hi

    lane_off = lax.iota(jnp.int32, LANES) * LANE_STRIDE
    d_idx.wait()

    n_acc = 8
    accs = tuple(jnp.zeros((LANES,), jnp.float32) for _ in range(n_acc))
    for r in range(BGROUP):
      buf = r % 2
      pend[buf].wait()
      pv = pred_b[buf]

      def sb_body(sb, carry, pv=pv, r=r):
        k = idx_v[r, pl.ds(sb * LANES, LANES)]
        koff = k * KSTRIDE + lane_off
        out = list(carry)
        for v in range(VOCAB):
          data = pv[v, pl.ds(sb * LANES, LANES)]
          w = plsc.load_gather(brep_v, [koff + v])
          out[v % n_acc] = out[v % n_acc] + w * data
        return tuple(out)

      accs = plsc.parallel_loop(0, sblk // LANES, carry=accs)(sb_body)
      if r + 2 < BGROUP:
        pend[buf] = start(r + 2, buf)

    total = accs[0]
    for v in range(1, n_acc):
      total = total + accs[v]
    acc_v[...] = total
    pltpu.sync_copy(acc_v, out_hbm.at[pl.ds(wid * LANES, LANES)])

  return blosum_sc


def _tc_body(yt_ref, yp_ref, b_ref, out_ref):
  i = pl.program_id(0)
  j = pl.program_id(1)
  iota_v = lax.broadcasted_iota(jnp.int32, (VOCAB, TC_SBLK), 0)
  s = jnp.float32(0.0)
  for r in range(BGROUP):
    k = yt_ref[r, :]
    onehot = (k[None, :] == iota_v).astype(jnp.float32)
    w = lax.dot_general(b_ref[...], onehot, (((0,), (0,)), ((), ())),
                        precision=lax.Precision.HIGHEST,
                        preferred_element_type=jnp.float32)
    s = s + jnp.sum(w * yp_ref[r])

  @pl.when((i == 0) & (j == 0))
  def _():
    out_ref[0, 0] = jnp.float32(0.0)

  out_ref[0, 0] += s


def _make_tc_kernel(tc_batch: int, seq: int, b_start: int):
  grid = (tc_batch // BGROUP, seq // TC_SBLK)
  bg0 = b_start // BGROUP
  return pl.pallas_call(
      _tc_body,
      grid=grid,
      in_specs=[
          pl.BlockSpec((BGROUP, TC_SBLK), lambda i, j: (i + bg0, j)),
          pl.BlockSpec((BGROUP, VOCAB, TC_SBLK),
                       lambda i, j: (i + bg0, 0, j)),
          pl.BlockSpec((VOCAB, VOCAB), lambda i, j: (0, 0)),
      ],
      out_specs=pl.BlockSpec(
          (1, 1), lambda i, j: (0, 0), memory_space=pltpu.SMEM),
      out_shape=jax.ShapeDtypeStruct((1, 1), jnp.float32),
      compiler_params=pltpu.CompilerParams(
          dimension_semantics=("arbitrary", "arbitrary")),
  )


@jax.jit
def kernel(y_true, y_pred, B):
  batch, seq = y_true.shape
  yp_t = jnp.transpose(y_pred, (0, 2, 1))
  sc_partials = _make_sc_kernel(SC_BATCH, seq)(y_true, yp_t, B.reshape(-1))
  tc_partial = _make_tc_kernel(batch - SC_BATCH, seq, SC_BATCH)(
      y_true, yp_t, B)
  return jnp.sum(sc_partials) + tc_partial[0, 0]
